# Initial kernel scaffold; baseline (speedup 1.0000x reference)
import jax
import jax.numpy as jnp
from jax.experimental import pallas as pl


def kernel(x, edge_index, W1, att_src1, att_dst1, b1, W2, att_src2, att_dst2, b2):
    raise NotImplementedError("write your pallas kernel here")


# speedup vs baseline: 33.4271x; 33.4271x over previous
import functools

import jax
import jax.numpy as jnp
from jax import lax
from jax.experimental import pallas as pl
from jax.experimental.pallas import tpu as pltpu
from jax.experimental.pallas import tpu_sc as plsc

N = 10000
E = 320000
NPAD = 10240
NACC = 10016
NC, NS, L = 2, 16, 16
NW = NC * NS
C = 128
KCH = 81
P = NW * C * KCH


def _round_block(n):
    return n


def _tc_prep_body(x_ref, w_ref, gs_ref, gd_ref, ts_ref, td_ref):
    h = jnp.dot(x_ref[...], w_ref[...], preferred_element_type=jnp.float32)
    ts_ref[...] = jnp.dot(h, gs_ref[...], preferred_element_type=jnp.float32)
    td_ref[...] = jnp.dot(h, gd_ref[...], preferred_element_type=jnp.float32)


def _tc_prep(xp, W, Gs, Gd):
    F = xp.shape[1]
    DS = Gs.shape[1]
    BN = 2048
    grid = (NPAD // BN,)
    return pl.pallas_call(
        _tc_prep_body,
        grid=grid,
        in_specs=[
            pl.BlockSpec((BN, F), lambda i: (i, 0)),
            pl.BlockSpec((F, W.shape[1]), lambda i: (0, 0)),
            pl.BlockSpec((W.shape[1], DS), lambda i: (0, 0)),
            pl.BlockSpec((W.shape[1], 16), lambda i: (0, 0)),
        ],
        out_specs=[
            pl.BlockSpec((BN, DS), lambda i: (i, 0)),
            pl.BlockSpec((BN, 16), lambda i: (i, 0)),
        ],
        out_shape=[
            jax.ShapeDtypeStruct((NPAD, DS), jnp.float32),
            jax.ShapeDtypeStruct((NPAD, 16), jnp.float32),
        ],
    )(xp, W, Gs, Gd)


def _tc_mid_body(ua_ref, ub_ref, da_ref, db_ref, r_ref, b_ref, w_ref,
                 gs_ref, gd_ref, ts_ref, td_ref):
    u = ua_ref[...] + ub_ref[...]
    den = jnp.dot(da_ref[...] + db_ref[...], r_ref[...],
                  preferred_element_type=jnp.float32)
    hpre = u / (den + 1e-16) + b_ref[...]
    h = jnp.where(hpre > 0, hpre, jnp.exp(jnp.minimum(hpre, 0.0)) - 1.0)
    h2 = jnp.dot(h, w_ref[...], preferred_element_type=jnp.float32)
    ts_ref[...] = jnp.dot(h2, gs_ref[...], preferred_element_type=jnp.float32)
    td_ref[...] = jnp.dot(h2, gd_ref[...], preferred_element_type=jnp.float32)


def _tc_mid(Ua, Ub, da, db, R, b1r, W2, Gs, Gd):
    DM = Ua.shape[1]
    DS = Gs.shape[1]
    F2 = W2.shape[1]
    BN = 2048
    grid = (NPAD // BN,)
    return pl.pallas_call(
        _tc_mid_body,
        grid=grid,
        in_specs=[
            pl.BlockSpec((BN, DM), lambda i: (i, 0)),
            pl.BlockSpec((BN, DM), lambda i: (i, 0)),
            pl.BlockSpec((BN, 16), lambda i: (i, 0)),
            pl.BlockSpec((BN, 16), lambda i: (i, 0)),
            pl.BlockSpec((16, DM), lambda i: (0, 0)),
            pl.BlockSpec((1, DM), lambda i: (0, 0)),
            pl.BlockSpec((DM, F2), lambda i: (0, 0)),
            pl.BlockSpec((F2, DS), lambda i: (0, 0)),
            pl.BlockSpec((F2, 16), lambda i: (0, 0)),
        ],
        out_specs=[
            pl.BlockSpec((BN, DS), lambda i: (i, 0)),
            pl.BlockSpec((BN, 16), lambda i: (i, 0)),
        ],
        out_shape=[
            jax.ShapeDtypeStruct((NPAD, DS), jnp.float32),
            jax.ShapeDtypeStruct((NPAD, 16), jnp.float32),
        ],
    )(Ua, Ub, da, db, R, b1r, W2, Gs, Gd)


def _tc_final_body(ua_ref, ub_ref, da_ref, db_ref, r_ref, b_ref, o_ref):
    u = ua_ref[...] + ub_ref[...]
    den = jnp.dot(da_ref[...] + db_ref[...], r_ref[...],
                  preferred_element_type=jnp.float32)
    o_ref[...] = u / (den + 1e-16) + b_ref[...]


def _tc_final(Ua, Ub, da, db, R2, b2r):
    DM = Ua.shape[1]
    BN = 2000
    grid = (N // BN,)
    return pl.pallas_call(
        _tc_final_body,
        grid=grid,
        in_specs=[
            pl.BlockSpec((BN, DM), lambda i: (i, 0)),
            pl.BlockSpec((BN, DM), lambda i: (i, 0)),
            pl.BlockSpec((BN, 16), lambda i: (i, 0)),
            pl.BlockSpec((BN, 16), lambda i: (i, 0)),
            pl.BlockSpec((16, DM), lambda i: (0, 0)),
            pl.BlockSpec((1, DM), lambda i: (0, 0)),
        ],
        out_specs=pl.BlockSpec((BN, DM), lambda i: (i, 0)),
        out_shape=jax.ShapeDtypeStruct((N, DM), jnp.float32),
    )(Ua, Ub, da, db, R2, b2r)


def _make_edge_pass(DS, DM, out_ch):
    NQ = DM // L
    ZR = NACC // NS
    sh = out_ch.bit_length() - 1

    def body(tsrc, tdst, src_hbm, dst_hbm, zdm, z16,
             ua, ub, da, db,
             src_v, dst_v, rows_v, drows_v, msg_v, t_v,
             u_sh, d_sh, sem1, sem2):
        cid = lax.axis_index("c")
        sid = lax.axis_index("s")
        w = cid * NS + sid

        pltpu.sync_copy(zdm, u_sh.at[pl.ds(sid * ZR, ZR)])
        pltpu.sync_copy(z16, d_sh.at[pl.ds(sid * ZR, ZR)])
        plsc.subcore_barrier()

        lane = lax.iota(jnp.int32, L)
        idx_vecs = [(lane + q * L) >> sh for q in range(NQ)]

        def chunk(k, carry):
            base = (w * KCH + k) * C
            pltpu.sync_copy(src_hbm.at[pl.ds(base, C)], src_v)
            pltpu.sync_copy(dst_hbm.at[pl.ds(base, C)], dst_v)
            g1 = pltpu.async_copy(tsrc.at[src_v], rows_v, sem1)
            g2 = pltpu.async_copy(tdst.at[dst_v], drows_v, sem2)
            g1.wait()
            g2.wait()

            def edge(e, carry2):
                asrc = rows_v[e, pl.ds(DS - L, L)]
                adst = drows_v[e, pl.ds(0, L)]
                s = asrc + adst
                s = jnp.where(s >= 0, s, 0.2 * s)
                t = jnp.exp(s)
                t_v[e, pl.ds(0, L)] = t
                for q in range(NQ):
                    tb = jnp.take_along_axis(t, idx_vecs[q], axis=0)
                    msg_v[e, pl.ds(q * L, L)] = (
                        rows_v[e, pl.ds(q * L, L)] * tb)
                return carry2

            lax.fori_loop(0, C, edge, 0, unroll=2)
            pltpu.sync_copy(msg_v, u_sh.at[dst_v], add=True)
            pltpu.sync_copy(t_v, d_sh.at[dst_v], add=True)
            return carry

        lax.fori_loop(0, KCH, chunk, 0)
        plsc.subcore_barrier()

        rs = pl.ds(sid * ZR, ZR)

        @pl.when(cid == 0)
        def _():
            pltpu.sync_copy(u_sh.at[rs], ua.at[rs])
            pltpu.sync_copy(d_sh.at[rs], da.at[rs])

        @pl.when(cid == 1)
        def _():
            pltpu.sync_copy(u_sh.at[rs], ub.at[rs])
            pltpu.sync_copy(d_sh.at[rs], db.at[rs])

    mesh = plsc.VectorSubcoreMesh(core_axis_name="c", subcore_axis_name="s",
                                  num_cores=NC, num_subcores=NS)
    return pl.kernel(
        body,
        out_type=[
            jax.ShapeDtypeStruct((NPAD, DM), jnp.float32),
            jax.ShapeDtypeStruct((NPAD, DM), jnp.float32),
            jax.ShapeDtypeStruct((NPAD, 16), jnp.float32),
            jax.ShapeDtypeStruct((NPAD, 16), jnp.float32),
        ],
        mesh=mesh,
        compiler_params=pltpu.CompilerParams(use_tc_tiling_on_sc=False),
        scratch_types=[
            pltpu.VMEM((C,), jnp.int32),
            pltpu.VMEM((C,), jnp.int32),
            pltpu.VMEM((C, DS), jnp.float32),
            pltpu.VMEM((C, 16), jnp.float32),
            pltpu.VMEM((C, DM), jnp.float32),
            pltpu.VMEM((C, 16), jnp.float32),
            pltpu.MemorySpace.VMEM_SHARED((NACC, DM), jnp.float32),
            pltpu.MemorySpace.VMEM_SHARED((NACC, 16), jnp.float32),
            pltpu.SemaphoreType.DMA,
            pltpu.SemaphoreType.DMA,
        ],
    )


_edge_pass_1 = _make_edge_pass(80, 64, 8)
_edge_pass_2 = _make_edge_pass(144, 128, 128)


def kernel(x, edge_index, W1, att_src1, att_dst1, b1, W2, att_src2,
           att_dst2, b2):
    f32 = jnp.float32
    heads1, ch1 = att_src1.shape
    d1 = heads1 * ch1

    xp = jnp.zeros((NPAD, x.shape[1]), f32).at[:N].set(x)
    loops = jnp.arange(N, dtype=jnp.int32)
    pad_idx = jnp.full((P - E - N,), N, dtype=jnp.int32)
    src = jnp.concatenate([edge_index[0].astype(jnp.int32), loops, pad_idx])
    dst = jnp.concatenate([edge_index[1].astype(jnp.int32), loops, pad_idx])

    eye_h = jnp.eye(heads1, dtype=f32)
    m_src1 = (att_src1[:, :, None] * eye_h[:, None, :]).reshape(d1, heads1)
    m_dst1 = (att_dst1[:, :, None] * eye_h[:, None, :]).reshape(d1, heads1)
    g1s = jnp.concatenate(
        [jnp.eye(d1, dtype=f32), m_src1, jnp.zeros((d1, 16 - heads1), f32)],
        axis=1)
    g1d = jnp.concatenate([m_dst1, jnp.zeros((d1, 16 - heads1), f32)], axis=1)

    d2 = W2.shape[1]
    g2s = jnp.concatenate(
        [jnp.eye(d2, dtype=f32), att_src2.T, jnp.zeros((d2, 15), f32)], axis=1)
    g2d = jnp.concatenate([att_dst2.T, jnp.zeros((d2, 15), f32)], axis=1)

    r1 = jnp.concatenate(
        [jnp.repeat(eye_h, ch1, axis=1), jnp.zeros((16 - heads1, d1), f32)],
        axis=0)
    r2 = jnp.zeros((16, d2), f32).at[0].set(1.0)

    z64 = jnp.zeros((NACC // NS, d1), f32)
    z128 = jnp.zeros((NACC // NS, d2), f32)
    z16 = jnp.zeros((NACC // NS, 16), f32)

    ts1, td1 = _tc_prep(xp, W1, g1s, g1d)
    ua1, ub1, da1, db1 = _edge_pass_1(ts1, td1, src, dst, z64, z16)

    ts2, td2 = _tc_mid(ua1, ub1, da1, db1, r1, b1.reshape(1, d1), W2,
                       g2s, g2d)
    ua2, ub2, da2, db2 = _edge_pass_2(ts2, td2, src, dst, z128, z16)

    return _tc_final(ua2[:N], ub2[:N], da2[:N], db2[:N], r2,
                     b2.reshape(1, d2))


# speedup vs baseline: 48.1596x; 1.4407x over previous
import functools

import jax
import jax.numpy as jnp
from jax import lax
from jax.experimental import pallas as pl
from jax.experimental.pallas import tpu as pltpu
from jax.experimental.pallas import tpu_sc as plsc

N = 10000
E = 320000
NPAD = 10240
NACC = 10016
NC, NS, L = 2, 16, 16
NW = NC * NS
C = 112
KCH = 94
KAL = KCH + 2
P = NW * C * KCH
PAL = NW * C * KAL


def _round_block(n):
    return n


def _tc_prep_body(x_ref, w_ref, gs_ref, gd_ref, th_ref, ts_ref, td_ref):
    h = jnp.dot(x_ref[...], w_ref[...], preferred_element_type=jnp.float32)
    th_ref[...] = h
    ts_ref[...] = jnp.dot(h, gs_ref[...], preferred_element_type=jnp.float32)
    td_ref[...] = jnp.dot(h, gd_ref[...], preferred_element_type=jnp.float32)


def _tc_prep(xp, W, Gs, Gd):
    F = xp.shape[1]
    DH = W.shape[1]
    BN = 2048
    grid = (NPAD // BN,)
    return pl.pallas_call(
        _tc_prep_body,
        grid=grid,
        in_specs=[
            pl.BlockSpec((BN, F), lambda i: (i, 0)),
            pl.BlockSpec((F, DH), lambda i: (0, 0)),
            pl.BlockSpec((DH, 16), lambda i: (0, 0)),
            pl.BlockSpec((DH, 16), lambda i: (0, 0)),
        ],
        out_specs=[
            pl.BlockSpec((BN, DH), lambda i: (i, 0)),
            pl.BlockSpec((BN, 16), lambda i: (i, 0)),
            pl.BlockSpec((BN, 16), lambda i: (i, 0)),
        ],
        out_shape=[
            jax.ShapeDtypeStruct((NPAD, DH), jnp.float32),
            jax.ShapeDtypeStruct((NPAD, 16), jnp.float32),
            jax.ShapeDtypeStruct((NPAD, 16), jnp.float32),
        ],
    )(xp, W, Gs, Gd)


def _tc_mid_body(ua_ref, ub_ref, da_ref, db_ref, r_ref, b_ref, w_ref,
                 gs_ref, gd_ref, th_ref, ts_ref, td_ref):
    u = ua_ref[...] + ub_ref[...]
    den = jnp.dot(da_ref[...] + db_ref[...], r_ref[...],
                  preferred_element_type=jnp.float32)
    hpre = u / (den + 1e-16) + b_ref[...]
    h = jnp.where(hpre > 0, hpre, jnp.exp(jnp.minimum(hpre, 0.0)) - 1.0)
    h2 = jnp.dot(h, w_ref[...], preferred_element_type=jnp.float32)
    th_ref[...] = h2
    ts_ref[...] = jnp.dot(h2, gs_ref[...], preferred_element_type=jnp.float32)
    td_ref[...] = jnp.dot(h2, gd_ref[...], preferred_element_type=jnp.float32)


def _tc_mid(Ua, Ub, da, db, R, b1r, W2, Gs, Gd):
    DM = Ua.shape[1]
    F2 = W2.shape[1]
    BN = 2048
    grid = (NPAD // BN,)
    return pl.pallas_call(
        _tc_mid_body,
        grid=grid,
        in_specs=[
            pl.BlockSpec((BN, DM), lambda i: (i, 0)),
            pl.BlockSpec((BN, DM), lambda i: (i, 0)),
            pl.BlockSpec((BN, 16), lambda i: (i, 0)),
            pl.BlockSpec((BN, 16), lambda i: (i, 0)),
            pl.BlockSpec((16, DM), lambda i: (0, 0)),
            pl.BlockSpec((1, DM), lambda i: (0, 0)),
            pl.BlockSpec((DM, F2), lambda i: (0, 0)),
            pl.BlockSpec((F2, 16), lambda i: (0, 0)),
            pl.BlockSpec((F2, 16), lambda i: (0, 0)),
        ],
        out_specs=[
            pl.BlockSpec((BN, F2), lambda i: (i, 0)),
            pl.BlockSpec((BN, 16), lambda i: (i, 0)),
            pl.BlockSpec((BN, 16), lambda i: (i, 0)),
        ],
        out_shape=[
            jax.ShapeDtypeStruct((NPAD, F2), jnp.float32),
            jax.ShapeDtypeStruct((NPAD, 16), jnp.float32),
            jax.ShapeDtypeStruct((NPAD, 16), jnp.float32),
        ],
    )(Ua, Ub, da, db, R, b1r, W2, Gs, Gd)


def _tc_final_body(ua_ref, ub_ref, da_ref, db_ref, r_ref, b_ref, o_ref):
    u = ua_ref[...] + ub_ref[...]
    den = jnp.dot(da_ref[...] + db_ref[...], r_ref[...],
                  preferred_element_type=jnp.float32)
    o_ref[...] = u / (den + 1e-16) + b_ref[...]


def _tc_final(Ua, Ub, da, db, R2, b2r):
    DM = Ua.shape[1]
    BN = 2000
    grid = (N // BN,)
    return pl.pallas_call(
        _tc_final_body,
        grid=grid,
        in_specs=[
            pl.BlockSpec((BN, DM), lambda i: (i, 0)),
            pl.BlockSpec((BN, DM), lambda i: (i, 0)),
            pl.BlockSpec((BN, 16), lambda i: (i, 0)),
            pl.BlockSpec((BN, 16), lambda i: (i, 0)),
            pl.BlockSpec((16, DM), lambda i: (0, 0)),
            pl.BlockSpec((1, DM), lambda i: (0, 0)),
        ],
        out_specs=pl.BlockSpec((BN, DM), lambda i: (i, 0)),
        out_shape=jax.ShapeDtypeStruct((N, DM), jnp.float32),
    )(Ua, Ub, da, db, R2, b2r)


def _make_edge_pass(DM, out_ch):
    NQ = DM // L
    ZR = NACC // NS
    sh = out_ch.bit_length() - 1

    def body(th, tas, tad, edges, zdm, z16,
             ua, ub, da, db,
             idx0, idx1, rows0, rows1, arows0, arows1, drows0, drows1,
             dsti0, dsti1, u_sh, d_sh,
             si0, si1, gr0, gr1, ga0, ga1, gd0, gd1, su0, su1, sd0, sd1):
        cid = lax.axis_index("c")
        sid = lax.axis_index("s")
        w = cid * NS + sid

        pltpu.sync_copy(zdm, u_sh.at[pl.ds(sid * ZR, ZR)])
        pltpu.sync_copy(z16, d_sh.at[pl.ds(sid * ZR, ZR)])
        plsc.subcore_barrier()

        lane = lax.iota(jnp.int32, L)
        idx_vecs = [(lane + q * L) >> sh for q in range(NQ)]

        bufs = (
            (idx0, rows0, arows0, drows0, dsti0, si0, gr0, ga0, su0, sd0),
            (idx1, rows1, arows1, drows1, dsti1, si1, gr1, ga1, su1, sd1),
        )
        gds = (gd0, gd1)

        def idx_copy(k, b):
            idx_v, _, _, _, _, si, *_ = bufs[b]
            return pltpu.make_async_copy(edges.at[w * KAL + k], idx_v, si)

        def gather_copies(b):
            idx_v, rows_v, arows_v, drows_v, _, _, gr, ga, _, _ = bufs[b]
            c1 = pltpu.make_async_copy(th.at[idx_v.at[0]], rows_v, gr)
            c2 = pltpu.make_async_copy(tas.at[idx_v.at[0]], arows_v, ga)
            c3 = pltpu.make_async_copy(tad.at[idx_v.at[1]], drows_v, gds[b])
            return c1, c2, c3

        def scatter_copies(b):
            _, rows_v, _, drows_v, dsti, _, _, _, su, sd = bufs[b]
            c1 = pltpu.make_async_copy(rows_v, u_sh.at[dsti], su)
            c2 = pltpu.make_async_copy(drows_v, d_sh.at[dsti], sd)
            return c1, c2

        def compute(b):
            idx_v, rows_v, arows_v, drows_v, dsti, *_ = bufs[b]
            for i in range(C // L):
                dsti[pl.ds(i * L, L)] = idx_v[1, pl.ds(i * L, L)]

            def edge(e, carry):
                asrc = arows_v[e, pl.ds(0, L)]
                adst = drows_v[e, pl.ds(0, L)]
                s = asrc + adst
                s = jnp.where(s >= 0, s, 0.2 * s)
                t = jnp.exp(s)
                drows_v[e, pl.ds(0, L)] = t
                for q in range(NQ):
                    tb = jnp.take_along_axis(t, idx_vecs[q], axis=0)
                    rows_v[e, pl.ds(q * L, L)] = (
                        rows_v[e, pl.ds(q * L, L)] * tb)
                return carry

            lax.fori_loop(0, C, edge, 0, unroll=2)

        def phase(k, b, first):
            nb = 1 - b
            idx_copy(k + 1, nb).wait()
            if not first:
                for c in scatter_copies(nb):
                    c.wait()
            for c in gather_copies(nb):
                c.start()
            for c in gather_copies(b):
                c.wait()
            compute(b)
            idx_copy(k + 2, b).start()
            s1, s2 = scatter_copies(b)
            s1.start(add=True)
            s2.start(add=True)

        pltpu.sync_copy(edges.at[w * KAL], idx0)
        for c in gather_copies(0):
            c.start()
        idx_copy(1, 1).start()
        phase(0, 0, True)
        phase(1, 1, False)

        def pair(k2, carry):
            phase(2 * k2, 0, False)
            phase(2 * k2 + 1, 1, False)
            return carry

        lax.fori_loop(1, KCH // 2, pair, 0)

        idx_copy(KCH + 1, 1).wait()
        for c in gather_copies(0):
            c.wait()
        for c in scatter_copies(1):
            c.wait()

        plsc.subcore_barrier()

        rs = pl.ds(sid * ZR, ZR)

        @pl.when(cid == 0)
        def _():
            pltpu.sync_copy(u_sh.at[rs], ua.at[rs])
            pltpu.sync_copy(d_sh.at[rs], da.at[rs])

        @pl.when(cid == 1)
        def _():
            pltpu.sync_copy(u_sh.at[rs], ub.at[rs])
            pltpu.sync_copy(d_sh.at[rs], db.at[rs])

    mesh = plsc.VectorSubcoreMesh(core_axis_name="c", subcore_axis_name="s",
                                  num_cores=NC, num_subcores=NS)
    return pl.kernel(
        body,
        out_type=[
            jax.ShapeDtypeStruct((NPAD, DM), jnp.float32),
            jax.ShapeDtypeStruct((NPAD, DM), jnp.float32),
            jax.ShapeDtypeStruct((NPAD, 16), jnp.float32),
            jax.ShapeDtypeStruct((NPAD, 16), jnp.float32),
        ],
        mesh=mesh,
        compiler_params=pltpu.CompilerParams(use_tc_tiling_on_sc=False),
        scratch_types=[
            pltpu.VMEM((2, C), jnp.int32),
            pltpu.VMEM((2, C), jnp.int32),
            pltpu.VMEM((C, DM), jnp.float32),
            pltpu.VMEM((C, DM), jnp.float32),
            pltpu.VMEM((C, 16), jnp.float32),
            pltpu.VMEM((C, 16), jnp.float32),
            pltpu.VMEM((C, 16), jnp.float32),
            pltpu.VMEM((C, 16), jnp.float32),
            pltpu.VMEM((C,), jnp.int32),
            pltpu.VMEM((C,), jnp.int32),
            pltpu.MemorySpace.VMEM_SHARED((NACC, DM), jnp.float32),
            pltpu.MemorySpace.VMEM_SHARED((NACC, 16), jnp.float32),
        ] + [pltpu.SemaphoreType.DMA] * 12,
    )


_edge_pass_1 = _make_edge_pass(64, 8)
_edge_pass_2 = _make_edge_pass(128, 128)


def kernel(x, edge_index, W1, att_src1, att_dst1, b1, W2, att_src2,
           att_dst2, b2):
    f32 = jnp.float32
    heads1, ch1 = att_src1.shape
    d1 = heads1 * ch1

    xp = jnp.zeros((NPAD, x.shape[1]), f32).at[:N].set(x)
    loops = jnp.arange(N, dtype=jnp.int32)
    pad_idx = jnp.full((P - E - N,), N, dtype=jnp.int32)
    src = jnp.concatenate([edge_index[0].astype(jnp.int32), loops, pad_idx])
    dst = jnp.concatenate([edge_index[1].astype(jnp.int32), loops, pad_idx])
    sd = jnp.stack([src.reshape(NW, KCH, C), dst.reshape(NW, KCH, C)],
                   axis=2)
    tail = jnp.full((NW, KAL - KCH, 2, C), N, dtype=jnp.int32)
    edges = jnp.concatenate([sd, tail], axis=1).reshape(NW * KAL, 2, C)

    eye_h = jnp.eye(heads1, dtype=f32)
    m_src1 = (att_src1[:, :, None] * eye_h[:, None, :]).reshape(d1, heads1)
    m_dst1 = (att_dst1[:, :, None] * eye_h[:, None, :]).reshape(d1, heads1)
    g1s = jnp.concatenate([m_src1, jnp.zeros((d1, 16 - heads1), f32)], axis=1)
    g1d = jnp.concatenate([m_dst1, jnp.zeros((d1, 16 - heads1), f32)], axis=1)

    d2 = W2.shape[1]
    g2s = jnp.concatenate([att_src2.T, jnp.zeros((d2, 15), f32)], axis=1)
    g2d = jnp.concatenate([att_dst2.T, jnp.zeros((d2, 15), f32)], axis=1)

    r1 = jnp.concatenate(
        [jnp.repeat(eye_h, ch1, axis=1), jnp.zeros((16 - heads1, d1), f32)],
        axis=0)
    r2 = jnp.zeros((16, d2), f32).at[0].set(1.0)

    z64 = jnp.zeros((NACC // NS, d1), f32)
    z128 = jnp.zeros((NACC // NS, d2), f32)
    z16 = jnp.zeros((NACC // NS, 16), f32)

    th1, tas1, tad1 = _tc_prep(xp, W1, g1s, g1d)
    ua1, ub1, da1, db1 = _edge_pass_1(th1, tas1, tad1, edges, z64, z16)

    th2, tas2, tad2 = _tc_mid(ua1, ub1, da1, db1, r1, b1.reshape(1, d1), W2,
                              g2s, g2d)
    ua2, ub2, da2, db2 = _edge_pass_2(th2, tas2, tad2, edges, z128, z16)

    return _tc_final(ua2[:N], ub2[:N], da2[:N], db2[:N], r2,
                     b2.reshape(1, d2))
